# baseline (speedup 1.0000x reference)
import functools

import jax
import jax.numpy as jnp
from jax import lax
from jax.experimental import pallas as pl
from jax.experimental.pallas import tpu as pltpu
from jax.experimental.pallas import tpu_sc as plsc

_D = 123
_DP = 128
_NW = 32
_CH = 320


def _table_body(emb_ref, t_ref):
    emb = emb_ref[...]

    def rbf(col, lo, hi, n, stride):
        steps = lax.broadcasted_iota(jnp.int32, (21, n), 1).astype(jnp.float32)
        mu = lo + steps * ((hi - lo) / (n - 1))
        d = (emb[:, col : col + 1] - mu) * (1.0 / stride)
        return jnp.exp(-(d * d))

    sig = jax.nn.sigmoid(emb[:, 3:6] * 6.0 - 3.0)
    t_ref[...] = jnp.concatenate(
        [
            rbf(0, -4.5, 4.5, 90, 0.1),
            rbf(1, 0.0, 2.2, 22, 0.1),
            rbf(2, -1.0, 1.0, 8, 0.25),
            sig,
            jnp.zeros((21, _DP - _D), jnp.float32),
        ],
        axis=1,
    )


def _build_sc_gather(n_rows):
    rows_per_w = n_rows // _NW
    n_chunks = rows_per_w // _CH
    mesh = plsc.VectorSubcoreMesh(core_axis_name="c", subcore_axis_name="s")

    @functools.partial(
        pl.kernel,
        mesh=mesh,
        out_type=jax.ShapeDtypeStruct((n_rows, _D), jnp.float32),
        scratch_types=[
            pltpu.VMEM((21, _DP), jnp.float32),
            pltpu.VMEM((_CH,), jnp.int32),
            pltpu.VMEM((_CH,), jnp.int32),
            pltpu.VMEM((_CH, _D), jnp.float32),
            pltpu.VMEM((_CH, _D), jnp.float32),
            pltpu.SemaphoreType.DMA,
            pltpu.SemaphoreType.DMA,
            pltpu.SemaphoreType.DMA,
            pltpu.SemaphoreType.DMA,
        ],
    )
    def sc_gather(
        x_hbm,
        tab_hbm,
        out_hbm,
        tab_v,
        idx_a,
        idx_b,
        out_a,
        out_b,
        osem_a,
        osem_b,
        isem_a,
        isem_b,
    ):
        wid = lax.axis_index("s") * 2 + lax.axis_index("c")
        base = wid * rows_per_w
        pltpu.sync_copy(tab_hbm, tab_v)
        bufs = (
            (idx_a, out_a, osem_a, isem_a),
            (idx_b, out_b, osem_b, isem_b),
        )
        n_pairs = n_chunks // 2
        pltpu.async_copy(x_hbm.at[pl.ds(base, _CH)], idx_a, isem_a)

        def pair(p, carry):
            for k in range(2):
                idx_v, out_v, osem, isem = bufs[k]
                nxt_idx, _, _, nxt_isem = bufs[1 - k]
                c = p * 2 + k
                b0 = base + c * _CH

                pltpu.make_async_copy(
                    x_hbm.at[pl.ds(b0, _CH)], idx_v, isem
                ).wait()

                @pl.when(jnp.logical_or(k == 0, p < n_pairs - 1))
                def _prefetch():
                    nb0 = base + jnp.minimum(c + 1, n_chunks - 1) * _CH
                    pltpu.async_copy(
                        x_hbm.at[pl.ds(nb0, _CH)], nxt_idx, nxt_isem
                    )

                @pl.when(p > 0)
                def _drain():
                    pltpu.make_async_copy(
                        out_v, out_hbm.at[pl.ds(b0, _CH)], osem
                    ).wait()

                @plsc.parallel_loop(0, _CH // 16, unroll=4)
                def grp(g):
                    bv = idx_v[pl.ds(g * 16, 16)]
                    for j in range(16):
                        b = bv[j]
                        i = g * 16 + j
                        for o in (0, 16, 32, 48, 64, 80, 96, 107):
                            out_v[i, pl.ds(o, 16)] = tab_v[b, pl.ds(o, 16)]

                pltpu.async_copy(out_v, out_hbm.at[pl.ds(b0, _CH)], osem)
            return carry

        lax.fori_loop(0, n_pairs, pair, 0)
        for k in range(2):
            idx_v, out_v, osem, isem = bufs[k]
            pltpu.make_async_copy(
                out_v, out_hbm.at[pl.ds(base, _CH)], osem
            ).wait()

    return sc_gather


def kernel(x, embedding):
    table = pl.pallas_call(
        _table_body,
        out_shape=jax.ShapeDtypeStruct((21, _DP), jnp.float32),
    )(embedding)
    x_flat = x.reshape(-1)
    return _build_sc_gather(x_flat.shape[0])(x_flat, table)


# speedup vs baseline: 1.6318x; 1.6318x over previous
import functools

import jax
import jax.numpy as jnp
from jax import lax
from jax.experimental import pallas as pl
from jax.experimental.pallas import tpu as pltpu
from jax.experimental.pallas import tpu_sc as plsc

_D = 123
_DP = 128
_NW = 32
_CH = 320


def _table_body(emb_ref, t_ref):
    emb = emb_ref[...]

    def rbf(col, lo, hi, n, stride):
        steps = lax.broadcasted_iota(jnp.int32, (21, n), 1).astype(jnp.float32)
        mu = lo + steps * ((hi - lo) / (n - 1))
        d = (emb[:, col : col + 1] - mu) * (1.0 / stride)
        return jnp.exp(-(d * d))

    sig = jax.nn.sigmoid(emb[:, 3:6] * 6.0 - 3.0)
    t_ref[...] = jnp.concatenate(
        [
            rbf(0, -4.5, 4.5, 90, 0.1),
            rbf(1, 0.0, 2.2, 22, 0.1),
            rbf(2, -1.0, 1.0, 8, 0.25),
            sig,
            jnp.zeros((21, _DP - _D), jnp.float32),
        ],
        axis=1,
    )


def _build_sc_gather(n_rows):
    rows_per_w = n_rows // _NW
    n_chunks = rows_per_w // _CH
    mesh = plsc.VectorSubcoreMesh(core_axis_name="c", subcore_axis_name="s")

    @functools.partial(
        pl.kernel,
        mesh=mesh,
        out_type=jax.ShapeDtypeStruct((n_rows, _D), jnp.float32),
        scratch_types=[
            pltpu.VMEM((21, _DP), jnp.float32),
            pltpu.VMEM((_CH,), jnp.int32),
            pltpu.VMEM((_CH,), jnp.int32),
            pltpu.VMEM((_CH, _D), jnp.float32),
            pltpu.VMEM((_CH, _D), jnp.float32),
            pltpu.SemaphoreType.DMA,
            pltpu.SemaphoreType.DMA,
            pltpu.SemaphoreType.DMA,
            pltpu.SemaphoreType.DMA,
        ],
    )
    def sc_gather(
        x_hbm,
        tab_hbm,
        out_hbm,
        tab_v,
        idx_a,
        idx_b,
        out_a,
        out_b,
        osem_a,
        osem_b,
        isem_a,
        isem_b,
    ):
        wid = lax.axis_index("s") * 2 + lax.axis_index("c")
        base = wid * rows_per_w
        pltpu.sync_copy(tab_hbm, tab_v)
        bufs = (
            (idx_a, out_a, osem_a, isem_a),
            (idx_b, out_b, osem_b, isem_b),
        )
        n_pairs = n_chunks // 2
        pltpu.async_copy(x_hbm.at[pl.ds(base, _CH)], idx_a, isem_a)

        def pair(p, carry):
            for k in range(2):
                idx_v, out_v, osem, isem = bufs[k]
                nxt_idx, _, _, nxt_isem = bufs[1 - k]
                c = p * 2 + k
                b0 = base + c * _CH

                pltpu.make_async_copy(
                    x_hbm.at[pl.ds(b0, _CH)], idx_v, isem
                ).wait()

                @pl.when(jnp.logical_or(k == 0, p < n_pairs - 1))
                def _prefetch():
                    nb0 = base + jnp.minimum(c + 1, n_chunks - 1) * _CH
                    pltpu.async_copy(
                        x_hbm.at[pl.ds(nb0, _CH)], nxt_idx, nxt_isem
                    )

                @pl.when(p > 0)
                def _drain():
                    pltpu.make_async_copy(
                        out_v, out_hbm.at[pl.ds(b0, _CH)], osem
                    ).wait()

                @plsc.parallel_loop(0, _CH // 16, unroll=2)
                def grp(g):
                    bv = idx_v[pl.ds(g * 16, 16)]
                    for j in range(16):
                        b = bv[j]
                        i = g * 16 + j
                        for o in (0, 16, 32, 48, 64, 80, 96, 107):
                            out_v[i, pl.ds(o, 16)] = tab_v[b, pl.ds(o, 16)]

                pltpu.async_copy(out_v, out_hbm.at[pl.ds(b0, _CH)], osem)
            return carry

        lax.fori_loop(0, n_pairs, pair, 0)
        for k in range(2):
            idx_v, out_v, osem, isem = bufs[k]
            pltpu.make_async_copy(
                out_v, out_hbm.at[pl.ds(base, _CH)], osem
            ).wait()

    return sc_gather


def kernel(x, embedding):
    table = pl.pallas_call(
        _table_body,
        out_shape=jax.ShapeDtypeStruct((21, _DP), jnp.float32),
    )(embedding)
    x_flat = x.reshape(-1)
    return _build_sc_gather(x_flat.shape[0])(x_flat, table)
